# Initial kernel scaffold; baseline (speedup 1.0000x reference)
import functools

import jax
import jax.numpy as jnp
from jax import lax
from jax.experimental import pallas as pl
from jax.experimental.pallas import tpu as pltpu
from jax.experimental.pallas import tpu_sc as plsc

N_NODES = 10000
N_EDGES = 160000
D = 256
HALF = 128

NC = 2
NS = 16
BATCH = 125
ROWS_PER_TILE = N_EDGES // NS // BATCH
ROWS_TOTAL = N_EDGES // BATCH
ZROWS = N_NODES // NS


def _emb_body(x_ref, we_ref, be_ref, wc_ref, out_ref):
    wc = we_ref[...] @ wc_ref[...]
    bc = be_ref[...] @ wc_ref[...]
    out_ref[...] = x_ref[...] @ wc + bc


def _make_table(x, W_emb, b_emb2, W_conv):
    R = 2000
    nb = N_NODES // R
    return pl.pallas_call(
        _emb_body,
        grid=(NC, nb),
        in_specs=[
            pl.BlockSpec((R, D), lambda c, r: (r, 0)),
            pl.BlockSpec((D, D), lambda c, r: (0, 0)),
            pl.BlockSpec((1, D), lambda c, r: (0, 0)),
            pl.BlockSpec((D, HALF), lambda c, r: (0, c)),
        ],
        out_specs=pl.BlockSpec((R, HALF), lambda c, r: (c * nb + r, 0)),
        out_shape=jax.ShapeDtypeStruct((2 * N_NODES, HALF), jnp.float32),
    )(x, W_emb, b_emb2, W_conv)


def _sc_body(table, srcs, dsts, zeros, out, src_v, dst_v, rows_v, acc, sem):
    c = lax.axis_index("c")
    s = lax.axis_index("s")
    base = s * ROWS_PER_TILE
    pltpu.sync_copy(srcs.at[pl.ds(c * ROWS_TOTAL + base, ROWS_PER_TILE)], src_v)
    pltpu.sync_copy(dsts.at[pl.ds(base, ROWS_PER_TILE)], dst_v)
    pltpu.sync_copy(zeros, acc.at[pl.ds(s * ZROWS, ZROWS)])
    plsc.subcore_barrier()

    def step(j, carry):
        pltpu.async_copy(table.at[src_v.at[j]], rows_v, sem).wait()
        pltpu.sync_copy(rows_v, acc.at[dst_v.at[j]], add=True)
        return carry

    lax.fori_loop(0, ROWS_PER_TILE, step, 0)
    plsc.subcore_barrier()
    pltpu.sync_copy(
        acc.at[pl.ds(s * ZROWS, ZROWS)],
        out.at[pl.ds(c * N_NODES + s * ZROWS, ZROWS)],
    )


_sc_segsum = functools.partial(
    pl.kernel,
    out_type=jax.ShapeDtypeStruct((2 * N_NODES, HALF), jnp.float32),
    mesh=plsc.VectorSubcoreMesh(core_axis_name="c", subcore_axis_name="s"),
    scratch_types=[
        pltpu.VMEM((ROWS_PER_TILE, BATCH), jnp.int32),
        pltpu.VMEM((ROWS_PER_TILE, BATCH), jnp.int32),
        pltpu.VMEM((BATCH, HALF), jnp.float32),
        pltpu.VMEM_SHARED((N_NODES, HALF), jnp.float32),
        pltpu.SemaphoreType.DMA,
    ],
)(_sc_body)


def _read_body(s_ref, bc_ref, wr_ref, br_ref, y_ref, hs_ref):
    hs = s_ref[...] + bc_ref[...]
    hs_ref[...] = hs
    y_ref[...] = jnp.tanh(hs) @ wr_ref[...] + br_ref[...]


def _readout(s_cat, b_conv2, W_read, b_read2):
    R = 2000
    nb = N_NODES // R
    return pl.pallas_call(
        _read_body,
        grid=(nb,),
        in_specs=[
            pl.BlockSpec((R, D), lambda r: (r, 0)),
            pl.BlockSpec((1, D), lambda r: (0, 0)),
            pl.BlockSpec((D, D), lambda r: (0, 0)),
            pl.BlockSpec((1, D), lambda r: (0, 0)),
        ],
        out_specs=[
            pl.BlockSpec((R, D), lambda r: (r, 0)),
            pl.BlockSpec((R, D), lambda r: (r, 0)),
        ],
        out_shape=[
            jax.ShapeDtypeStruct((N_NODES, D), jnp.float32),
            jax.ShapeDtypeStruct((N_NODES, D), jnp.float32),
        ],
    )(s_cat, b_conv2, W_read, b_read2)


def kernel(x, edge_index, W_emb, b_emb, W_conv, b_conv, W_read, b_read):
    ei = edge_index.astype(jnp.int32)
    src, dst = ei[0], ei[1]
    srcs2 = jnp.concatenate([src, src + N_NODES]).reshape(NC * ROWS_TOTAL, BATCH)
    dst2 = dst.reshape(ROWS_TOTAL, BATCH)
    zeros = jnp.zeros((ZROWS, HALF), jnp.float32)

    table = _make_table(x, W_emb, b_emb.reshape(1, D), W_conv)
    s = _sc_segsum(table, srcs2, dst2, zeros)
    s_cat = jnp.concatenate([s[:N_NODES], s[N_NODES:]], axis=1)
    y, h_state = _readout(
        s_cat, b_conv.reshape(1, D), W_read, b_read.reshape(1, D)
    )
    return (y, h_state)


# speedup vs baseline: 5.5018x; 5.5018x over previous
import functools

import jax
import jax.numpy as jnp
from jax import lax
from jax.experimental import pallas as pl
from jax.experimental.pallas import tpu as pltpu
from jax.experimental.pallas import tpu_sc as plsc

N_NODES = 10000
N_EDGES = 160000
D = 256
HALF = 128

NC = 2
NS = 16
BATCH = 125
ROWS_PER_TILE = N_EDGES // NS // BATCH
ROWS_TOTAL = N_EDGES // BATCH
N_PAD = 10240
ZROWS = N_PAD // NS


def _emb_body(x_ref, we_ref, be_ref, wc_ref, out_ref):
    wc = we_ref[...] @ wc_ref[...]
    bc = be_ref[...] @ wc_ref[...]
    out_ref[...] = x_ref[...] @ wc + bc


def _make_table(x, W_emb, b_emb2, W_conv):
    R = 2000
    nb = N_NODES // R
    return pl.pallas_call(
        _emb_body,
        grid=(NC, nb),
        in_specs=[
            pl.BlockSpec((R, D), lambda c, r: (r, 0)),
            pl.BlockSpec((D, D), lambda c, r: (0, 0)),
            pl.BlockSpec((1, D), lambda c, r: (0, 0)),
            pl.BlockSpec((D, HALF), lambda c, r: (0, c)),
        ],
        out_specs=pl.BlockSpec((R, HALF), lambda c, r: (c * nb + r, 0)),
        out_shape=jax.ShapeDtypeStruct((2 * N_NODES, HALF), jnp.float32),
    )(x, W_emb, b_emb2, W_conv)


def _sc_body(table, srcs, dsts, zeros, out, src_v, dst_v, rows_v, acc, sem):
    c = lax.axis_index("c")
    s = lax.axis_index("s")
    base = s * ROWS_PER_TILE
    pltpu.sync_copy(srcs.at[pl.ds(c * ROWS_TOTAL + base, ROWS_PER_TILE)], src_v)
    pltpu.sync_copy(dsts.at[pl.ds(base, ROWS_PER_TILE)], dst_v)
    pltpu.sync_copy(zeros, acc.at[pl.ds(s * ZROWS, ZROWS)])
    plsc.subcore_barrier()

    def step(j, carry):
        pltpu.async_copy(table.at[src_v.at[j]], rows_v, sem).wait()
        pltpu.sync_copy(rows_v, acc.at[dst_v.at[j]], add=True)
        return carry

    lax.fori_loop(0, ROWS_PER_TILE, step, 0)
    plsc.subcore_barrier()
    pltpu.sync_copy(
        acc.at[pl.ds(s * ZROWS, ZROWS)],
        out.at[pl.ds(c * N_PAD + s * ZROWS, ZROWS)],
    )


_sc_segsum = functools.partial(
    pl.kernel,
    out_type=jax.ShapeDtypeStruct((2 * N_PAD, HALF), jnp.float32),
    mesh=plsc.VectorSubcoreMesh(core_axis_name="c", subcore_axis_name="s"),
    scratch_types=[
        pltpu.VMEM((ROWS_PER_TILE, BATCH), jnp.int32),
        pltpu.VMEM((ROWS_PER_TILE, BATCH), jnp.int32),
        pltpu.VMEM((BATCH, HALF), jnp.float32),
        pltpu.VMEM_SHARED((N_PAD, HALF), jnp.float32),
        pltpu.SemaphoreType.DMA,
    ],
)(_sc_body)


def _read_body(s_ref, bc_ref, wr_ref, br_ref, y_ref, hs_ref):
    hs = s_ref[...] + bc_ref[...]
    hs_ref[...] = hs
    y_ref[...] = jnp.tanh(hs) @ wr_ref[...] + br_ref[...]


def _readout(s_cat, b_conv2, W_read, b_read2):
    R = 2000
    nb = N_NODES // R
    return pl.pallas_call(
        _read_body,
        grid=(nb,),
        in_specs=[
            pl.BlockSpec((R, D), lambda r: (r, 0)),
            pl.BlockSpec((1, D), lambda r: (0, 0)),
            pl.BlockSpec((D, D), lambda r: (0, 0)),
            pl.BlockSpec((1, D), lambda r: (0, 0)),
        ],
        out_specs=[
            pl.BlockSpec((R, D), lambda r: (r, 0)),
            pl.BlockSpec((R, D), lambda r: (r, 0)),
        ],
        out_shape=[
            jax.ShapeDtypeStruct((N_NODES, D), jnp.float32),
            jax.ShapeDtypeStruct((N_NODES, D), jnp.float32),
        ],
    )(s_cat, b_conv2, W_read, b_read2)


def kernel(x, edge_index, W_emb, b_emb, W_conv, b_conv, W_read, b_read):
    ei = edge_index.astype(jnp.int32)
    src, dst = ei[0], ei[1]
    srcs2 = jnp.concatenate([src, src + N_NODES]).reshape(NC * ROWS_TOTAL, BATCH)
    dst2 = dst.reshape(ROWS_TOTAL, BATCH)
    zeros = jnp.zeros((ZROWS, HALF), jnp.float32)

    table = _make_table(x, W_emb, b_emb.reshape(1, D), W_conv)
    s = _sc_segsum(table, srcs2, dst2, zeros)
    s_cat = jnp.concatenate(
        [s[:N_NODES], s[N_PAD:N_PAD + N_NODES]], axis=1
    )
    y, h_state = _readout(
        s_cat, b_conv.reshape(1, D), W_read, b_read.reshape(1, D)
    )
    return (y, h_state)


# speedup vs baseline: 8.0288x; 1.4593x over previous
import functools

import jax
import jax.numpy as jnp
from jax import lax
from jax.experimental import pallas as pl
from jax.experimental.pallas import tpu as pltpu
from jax.experimental.pallas import tpu_sc as plsc

N_NODES = 10000
N_EDGES = 160000
D = 256
HALF = 128

NC = 2
NS = 16
BATCH = 125
ROWS_PER_TILE = N_EDGES // NS // BATCH
ROWS_TOTAL = N_EDGES // BATCH
DBLK = 16
NBLK = ROWS_PER_TILE // DBLK
ZROWS = 624
ZTAIL = N_NODES - NS * ZROWS


def _emb_body(x_ref, we_ref, be_ref, wc_ref, out_ref):
    wc = we_ref[...] @ wc_ref[...]
    bc = be_ref[...] @ wc_ref[...]
    out_ref[...] = x_ref[...] @ wc + bc


def _make_table(x, W_emb, b_emb2, W_conv):
    R = 2000
    nb = N_NODES // R
    return pl.pallas_call(
        _emb_body,
        grid=(NC, nb),
        in_specs=[
            pl.BlockSpec((R, D), lambda c, r: (r, 0)),
            pl.BlockSpec((D, D), lambda c, r: (0, 0)),
            pl.BlockSpec((1, D), lambda c, r: (0, 0)),
            pl.BlockSpec((D, HALF), lambda c, r: (0, c)),
        ],
        out_specs=pl.BlockSpec((R, HALF), lambda c, r: (c * nb + r, 0)),
        out_shape=jax.ShapeDtypeStruct((2 * N_NODES, HALF), jnp.float32),
    )(x, W_emb, b_emb2, W_conv)


def _sc_body(table, srcs, dsts, zeros, out0, out1,
             src_v, dstage, bufs, acc, semg0, semg1, semd0, semd1):
    c = lax.axis_index("c")
    s = lax.axis_index("s")
    base = s * ROWS_PER_TILE
    pltpu.sync_copy(srcs.at[pl.ds(c * ROWS_TOTAL + base, ROWS_PER_TILE)], src_v)
    pltpu.sync_copy(dsts.at[pl.ds(base, DBLK)], dstage.at[0])
    pltpu.sync_copy(zeros, acc.at[pl.ds(s * ZROWS, ZROWS)])

    @pl.when(s == NS - 1)
    def _():
        pltpu.sync_copy(zeros.at[pl.ds(0, ZTAIL)],
                        acc.at[pl.ds(NS * ZROWS, ZTAIL)])

    plsc.subcore_barrier()

    pltpu.async_copy(table.at[src_v.at[0]], bufs.at[0], semg0)

    def step(i, carry):
        p = i % 2
        q = (i + 1) % 2
        blk = i // DBLK
        bp = blk % 2

        @pl.when((i % DBLK == 0) & (blk < NBLK - 1))
        def _():
            nxt = dsts.at[pl.ds(base + (blk + 1) * DBLK, DBLK)]

            @pl.when(bp == 0)
            def _():
                pltpu.async_copy(nxt, dstage.at[1], semd1)

            @pl.when(bp == 1)
            def _():
                pltpu.async_copy(nxt, dstage.at[0], semd0)

        @pl.when((i % DBLK == 0) & (i > 0))
        def _():
            cur = dsts.at[pl.ds(base + blk * DBLK, DBLK)]

            @pl.when(bp == 0)
            def _():
                pltpu.make_async_copy(cur, dstage.at[0], semd0).wait()

            @pl.when(bp == 1)
            def _():
                pltpu.make_async_copy(cur, dstage.at[1], semd1).wait()

        @pl.when(i < ROWS_PER_TILE - 1)
        def _():
            gsrc = table.at[src_v.at[i + 1]]

            @pl.when(q == 0)
            def _():
                pltpu.async_copy(gsrc, bufs.at[0], semg0)

            @pl.when(q == 1)
            def _():
                pltpu.async_copy(gsrc, bufs.at[1], semg1)

        gcur = table.at[src_v.at[i]]

        @pl.when(p == 0)
        def _():
            pltpu.make_async_copy(gcur, bufs.at[0], semg0).wait()

        @pl.when(p == 1)
        def _():
            pltpu.make_async_copy(gcur, bufs.at[1], semg1).wait()

        pltpu.sync_copy(bufs.at[p], acc.at[dstage.at[bp, i % DBLK]], add=True)
        return carry

    lax.fori_loop(0, ROWS_PER_TILE, step, 0)
    plsc.subcore_barrier()

    @pl.when(c == 0)
    def _():
        pltpu.sync_copy(acc.at[pl.ds(s * ZROWS, ZROWS)],
                        out0.at[pl.ds(s * ZROWS, ZROWS)])

        @pl.when(s == NS - 1)
        def _():
            pltpu.sync_copy(acc.at[pl.ds(NS * ZROWS, ZTAIL)],
                            out0.at[pl.ds(NS * ZROWS, ZTAIL)])

    @pl.when(c == 1)
    def _():
        pltpu.sync_copy(acc.at[pl.ds(s * ZROWS, ZROWS)],
                        out1.at[pl.ds(s * ZROWS, ZROWS)])

        @pl.when(s == NS - 1)
        def _():
            pltpu.sync_copy(acc.at[pl.ds(NS * ZROWS, ZTAIL)],
                            out1.at[pl.ds(NS * ZROWS, ZTAIL)])


_sc_segsum = functools.partial(
    pl.kernel,
    out_type=(
        jax.ShapeDtypeStruct((N_NODES, HALF), jnp.float32),
        jax.ShapeDtypeStruct((N_NODES, HALF), jnp.float32),
    ),
    mesh=plsc.VectorSubcoreMesh(core_axis_name="c", subcore_axis_name="s"),
    scratch_types=[
        pltpu.VMEM((ROWS_PER_TILE, BATCH), jnp.int32),
        pltpu.VMEM((2, DBLK, BATCH), jnp.int32),
        pltpu.VMEM((2, BATCH, HALF), jnp.float32),
        pltpu.VMEM_SHARED((N_NODES, HALF), jnp.float32),
        pltpu.SemaphoreType.DMA,
        pltpu.SemaphoreType.DMA,
        pltpu.SemaphoreType.DMA,
        pltpu.SemaphoreType.DMA,
    ],
)(_sc_body)


def _read_body(s0_ref, s1_ref, bc_ref, wr_ref, br_ref, y_ref, hs_ref):
    hs = jnp.concatenate([s0_ref[...], s1_ref[...]], axis=1) + bc_ref[...]
    hs_ref[...] = hs
    y_ref[...] = jnp.tanh(hs) @ wr_ref[...] + br_ref[...]


def _readout(s0, s1, b_conv2, W_read, b_read2):
    R = 2000
    nb = N_NODES // R
    return pl.pallas_call(
        _read_body,
        grid=(nb,),
        in_specs=[
            pl.BlockSpec((R, HALF), lambda r: (r, 0)),
            pl.BlockSpec((R, HALF), lambda r: (r, 0)),
            pl.BlockSpec((1, D), lambda r: (0, 0)),
            pl.BlockSpec((D, D), lambda r: (0, 0)),
            pl.BlockSpec((1, D), lambda r: (0, 0)),
        ],
        out_specs=[
            pl.BlockSpec((R, D), lambda r: (r, 0)),
            pl.BlockSpec((R, D), lambda r: (r, 0)),
        ],
        out_shape=[
            jax.ShapeDtypeStruct((N_NODES, D), jnp.float32),
            jax.ShapeDtypeStruct((N_NODES, D), jnp.float32),
        ],
    )(s0, s1, b_conv2, W_read, b_read2)


def kernel(x, edge_index, W_emb, b_emb, W_conv, b_conv, W_read, b_read):
    ei = edge_index.astype(jnp.int32)
    src, dst = ei[0], ei[1]
    srcs2 = jnp.concatenate([src, src + N_NODES]).reshape(NC * ROWS_TOTAL, BATCH)
    dst2 = dst.reshape(ROWS_TOTAL, BATCH)
    zeros = jnp.zeros((ZROWS, HALF), jnp.float32)

    table = _make_table(x, W_emb, b_emb.reshape(1, D), W_conv)
    s0, s1 = _sc_segsum(table, srcs2, dst2, zeros)
    y, h_state = _readout(
        s0, s1, b_conv.reshape(1, D), W_read, b_read.reshape(1, D)
    )
    return (y, h_state)


# speedup vs baseline: 8.0481x; 1.0024x over previous
import functools

import jax
import jax.numpy as jnp
from jax import lax
from jax.experimental import pallas as pl
from jax.experimental.pallas import tpu as pltpu
from jax.experimental.pallas import tpu_sc as plsc

N_NODES = 10000
N_EDGES = 160000
D = 256
HALF = 128

NC = 2
NS = 16
BATCH = 125
ROWS_PER_TILE = N_EDGES // NS // BATCH
ROWS_TOTAL = N_EDGES // BATCH
DBLK = 16
NBLK = ROWS_PER_TILE // DBLK
ZROWS = 624
ZTAIL = N_NODES - NS * ZROWS


def _emb_body(x_ref, we_ref, be_ref, wc_ref, out_ref):
    wc = we_ref[...] @ wc_ref[...]
    bc = be_ref[...] @ wc_ref[...]
    out_ref[...] = x_ref[...] @ wc + bc


def _make_table(x, W_emb, b_emb2, W_conv):
    R = 2000
    nb = N_NODES // R
    return pl.pallas_call(
        _emb_body,
        grid=(NC, nb),
        in_specs=[
            pl.BlockSpec((R, D), lambda c, r: (r, 0)),
            pl.BlockSpec((D, D), lambda c, r: (0, 0)),
            pl.BlockSpec((1, D), lambda c, r: (0, 0)),
            pl.BlockSpec((D, HALF), lambda c, r: (0, c)),
        ],
        out_specs=pl.BlockSpec((R, HALF), lambda c, r: (c * nb + r, 0)),
        out_shape=jax.ShapeDtypeStruct((2 * N_NODES, HALF), jnp.float32),
    )(x, W_emb, b_emb2, W_conv)


def _sc_body(table, srcs, dsts, zeros, out0, out1,
             src_v, dstage, bufs, acc,
             semg0, semg1, semd0, semd1, sems0, sems1):
    c = lax.axis_index("c")
    s = lax.axis_index("s")
    base = s * ROWS_PER_TILE
    pltpu.sync_copy(srcs.at[pl.ds(c * ROWS_TOTAL + base, ROWS_PER_TILE)], src_v)
    pltpu.sync_copy(dsts.at[pl.ds(base, DBLK)], dstage.at[0])
    pltpu.sync_copy(zeros, acc.at[pl.ds(s * ZROWS, ZROWS)])

    @pl.when(s == NS - 1)
    def _():
        pltpu.sync_copy(zeros.at[pl.ds(0, ZTAIL)],
                        acc.at[pl.ds(NS * ZROWS, ZTAIL)])

    plsc.subcore_barrier()

    pltpu.async_copy(table.at[src_v.at[0]], bufs.at[0], semg0)

    def step(i, carry):
        p = i % 2
        q = (i + 1) % 2
        blk = i // DBLK
        bp = blk % 2

        @pl.when(i >= 1)
        def _():
            dummy = acc.at[dstage.at[bp, 0]]

            @pl.when(q == 0)
            def _():
                pltpu.make_async_copy(bufs.at[0], dummy, sems0).wait()

            @pl.when(q == 1)
            def _():
                pltpu.make_async_copy(bufs.at[1], dummy, sems1).wait()

        @pl.when((i % DBLK == 0) & (blk < NBLK - 1))
        def _():
            nxt = dsts.at[pl.ds(base + (blk + 1) * DBLK, DBLK)]

            @pl.when(bp == 0)
            def _():
                pltpu.async_copy(nxt, dstage.at[1], semd1)

            @pl.when(bp == 1)
            def _():
                pltpu.async_copy(nxt, dstage.at[0], semd0)

        @pl.when((i % DBLK == 0) & (i > 0))
        def _():
            cur = dsts.at[pl.ds(base + blk * DBLK, DBLK)]

            @pl.when(bp == 0)
            def _():
                pltpu.make_async_copy(cur, dstage.at[0], semd0).wait()

            @pl.when(bp == 1)
            def _():
                pltpu.make_async_copy(cur, dstage.at[1], semd1).wait()

        @pl.when(i < ROWS_PER_TILE - 1)
        def _():
            gsrc = table.at[src_v.at[i + 1]]

            @pl.when(q == 0)
            def _():
                pltpu.async_copy(gsrc, bufs.at[0], semg0)

            @pl.when(q == 1)
            def _():
                pltpu.async_copy(gsrc, bufs.at[1], semg1)

        gcur = table.at[src_v.at[i]]

        @pl.when(p == 0)
        def _():
            pltpu.make_async_copy(gcur, bufs.at[0], semg0).wait()

        @pl.when(p == 1)
        def _():
            pltpu.make_async_copy(gcur, bufs.at[1], semg1).wait()

        sdst = acc.at[dstage.at[bp, i % DBLK]]

        @pl.when(p == 0)
        def _():
            pltpu.async_copy(bufs.at[0], sdst, sems0, add=True)

        @pl.when(p == 1)
        def _():
            pltpu.async_copy(bufs.at[1], sdst, sems1, add=True)

        return carry

    lax.fori_loop(0, ROWS_PER_TILE, step, 0)
    lastd = acc.at[dstage.at[(NBLK - 1) % 2, 0]]
    pltpu.make_async_copy(bufs.at[1], lastd, sems1).wait()
    plsc.subcore_barrier()

    @pl.when(c == 0)
    def _():
        pltpu.sync_copy(acc.at[pl.ds(s * ZROWS, ZROWS)],
                        out0.at[pl.ds(s * ZROWS, ZROWS)])

        @pl.when(s == NS - 1)
        def _():
            pltpu.sync_copy(acc.at[pl.ds(NS * ZROWS, ZTAIL)],
                            out0.at[pl.ds(NS * ZROWS, ZTAIL)])

    @pl.when(c == 1)
    def _():
        pltpu.sync_copy(acc.at[pl.ds(s * ZROWS, ZROWS)],
                        out1.at[pl.ds(s * ZROWS, ZROWS)])

        @pl.when(s == NS - 1)
        def _():
            pltpu.sync_copy(acc.at[pl.ds(NS * ZROWS, ZTAIL)],
                            out1.at[pl.ds(NS * ZROWS, ZTAIL)])


_sc_segsum = functools.partial(
    pl.kernel,
    out_type=(
        jax.ShapeDtypeStruct((N_NODES, HALF), jnp.float32),
        jax.ShapeDtypeStruct((N_NODES, HALF), jnp.float32),
    ),
    mesh=plsc.VectorSubcoreMesh(core_axis_name="c", subcore_axis_name="s"),
    scratch_types=[
        pltpu.VMEM((ROWS_PER_TILE, BATCH), jnp.int32),
        pltpu.VMEM((2, DBLK, BATCH), jnp.int32),
        pltpu.VMEM((2, BATCH, HALF), jnp.float32),
        pltpu.VMEM_SHARED((N_NODES, HALF), jnp.float32),
        pltpu.SemaphoreType.DMA,
        pltpu.SemaphoreType.DMA,
        pltpu.SemaphoreType.DMA,
        pltpu.SemaphoreType.DMA,
        pltpu.SemaphoreType.DMA,
        pltpu.SemaphoreType.DMA,
    ],
)(_sc_body)


def _read_body(s0_ref, s1_ref, bc_ref, wr_ref, br_ref, y_ref, hs_ref):
    hs = jnp.concatenate([s0_ref[...], s1_ref[...]], axis=1) + bc_ref[...]
    hs_ref[...] = hs
    y_ref[...] = jnp.tanh(hs) @ wr_ref[...] + br_ref[...]


def _readout(s0, s1, b_conv2, W_read, b_read2):
    R = 2000
    nb = N_NODES // R
    return pl.pallas_call(
        _read_body,
        grid=(nb,),
        in_specs=[
            pl.BlockSpec((R, HALF), lambda r: (r, 0)),
            pl.BlockSpec((R, HALF), lambda r: (r, 0)),
            pl.BlockSpec((1, D), lambda r: (0, 0)),
            pl.BlockSpec((D, D), lambda r: (0, 0)),
            pl.BlockSpec((1, D), lambda r: (0, 0)),
        ],
        out_specs=[
            pl.BlockSpec((R, D), lambda r: (r, 0)),
            pl.BlockSpec((R, D), lambda r: (r, 0)),
        ],
        out_shape=[
            jax.ShapeDtypeStruct((N_NODES, D), jnp.float32),
            jax.ShapeDtypeStruct((N_NODES, D), jnp.float32),
        ],
    )(s0, s1, b_conv2, W_read, b_read2)


def kernel(x, edge_index, W_emb, b_emb, W_conv, b_conv, W_read, b_read):
    ei = edge_index.astype(jnp.int32)
    src, dst = ei[0], ei[1]
    srcs2 = jnp.concatenate([src, src + N_NODES]).reshape(NC * ROWS_TOTAL, BATCH)
    dst2 = dst.reshape(ROWS_TOTAL, BATCH)
    zeros = jnp.zeros((ZROWS, HALF), jnp.float32)

    table = _make_table(x, W_emb, b_emb.reshape(1, D), W_conv)
    s0, s1 = _sc_segsum(table, srcs2, dst2, zeros)
    y, h_state = _readout(
        s0, s1, b_conv.reshape(1, D), W_read, b_read.reshape(1, D)
    )
    return (y, h_state)


# speedup vs baseline: 8.3764x; 1.0408x over previous
import functools

import jax
import jax.numpy as jnp
from jax import lax
from jax.experimental import pallas as pl
from jax.experimental.pallas import tpu as pltpu
from jax.experimental.pallas import tpu_sc as plsc

N_NODES = 10000
N_EDGES = 160000
D = 256
HALF = 128

NC = 2
NS = 16
BATCH = 125
ROWS_PER_TILE = N_EDGES // NS // BATCH
ROWS_TOTAL = N_EDGES // BATCH
DBLK = 16
NBLK = ROWS_PER_TILE // DBLK
ZROWS = 624
ZTAIL = N_NODES - NS * ZROWS


def _emb_body(x_ref, we_ref, be_ref, wc_ref, o0_ref, o1_ref):
    wc = we_ref[...] @ wc_ref[...]
    bc = be_ref[...] @ wc_ref[...]
    t = x_ref[...] @ wc + bc
    o0_ref[...] = t[:, :HALF]
    o1_ref[...] = t[:, HALF:]


def _make_table(x, W_emb, b_emb2, W_conv):
    R = 2000
    nb = N_NODES // R
    return pl.pallas_call(
        _emb_body,
        grid=(nb,),
        in_specs=[
            pl.BlockSpec((R, D), lambda r: (r, 0)),
            pl.BlockSpec((D, D), lambda r: (0, 0)),
            pl.BlockSpec((1, D), lambda r: (0, 0)),
            pl.BlockSpec((D, D), lambda r: (0, 0)),
        ],
        out_specs=[
            pl.BlockSpec((R, HALF), lambda r: (r, 0)),
            pl.BlockSpec((R, HALF), lambda r: (r, 0)),
        ],
        out_shape=[
            jax.ShapeDtypeStruct((N_NODES, HALF), jnp.float32),
            jax.ShapeDtypeStruct((N_NODES, HALF), jnp.float32),
        ],
    )(x, W_emb, b_emb2, W_conv)


def _sc_pipeline(table, dsts, base, src_v, dstage, bufs, acc,
                 semg0, semg1, semd0, semd1, sems0, sems1):
    pltpu.async_copy(table.at[src_v.at[0]], bufs.at[0], semg0)

    def step(i, carry):
        p = i % 2
        q = (i + 1) % 2
        blk = i // DBLK
        bp = blk % 2

        @pl.when(i >= 1)
        def _():
            dummy = acc.at[dstage.at[bp, 0]]

            @pl.when(q == 0)
            def _():
                pltpu.make_async_copy(bufs.at[0], dummy, sems0).wait()

            @pl.when(q == 1)
            def _():
                pltpu.make_async_copy(bufs.at[1], dummy, sems1).wait()

        @pl.when((i % DBLK == 0) & (blk < NBLK - 1))
        def _():
            nxt = dsts.at[pl.ds(base + (blk + 1) * DBLK, DBLK)]

            @pl.when(bp == 0)
            def _():
                pltpu.async_copy(nxt, dstage.at[1], semd1)

            @pl.when(bp == 1)
            def _():
                pltpu.async_copy(nxt, dstage.at[0], semd0)

        @pl.when((i % DBLK == 0) & (i > 0))
        def _():
            cur = dsts.at[pl.ds(base + blk * DBLK, DBLK)]

            @pl.when(bp == 0)
            def _():
                pltpu.make_async_copy(cur, dstage.at[0], semd0).wait()

            @pl.when(bp == 1)
            def _():
                pltpu.make_async_copy(cur, dstage.at[1], semd1).wait()

        @pl.when(i < ROWS_PER_TILE - 1)
        def _():
            gsrc = table.at[src_v.at[i + 1]]

            @pl.when(q == 0)
            def _():
                pltpu.async_copy(gsrc, bufs.at[0], semg0)

            @pl.when(q == 1)
            def _():
                pltpu.async_copy(gsrc, bufs.at[1], semg1)

        gcur = table.at[src_v.at[i]]

        @pl.when(p == 0)
        def _():
            pltpu.make_async_copy(gcur, bufs.at[0], semg0).wait()

        @pl.when(p == 1)
        def _():
            pltpu.make_async_copy(gcur, bufs.at[1], semg1).wait()

        sdst = acc.at[dstage.at[bp, i % DBLK]]

        @pl.when(p == 0)
        def _():
            pltpu.async_copy(bufs.at[0], sdst, sems0, add=True)

        @pl.when(p == 1)
        def _():
            pltpu.async_copy(bufs.at[1], sdst, sems1, add=True)

        return carry

    lax.fori_loop(0, ROWS_PER_TILE, step, 0)
    lastd = acc.at[dstage.at[(NBLK - 1) % 2, 0]]
    pltpu.make_async_copy(bufs.at[1], lastd, sems1).wait()


def _sc_body(table0, table1, srcs, dsts, zeros, out0, out1,
             src_v, dstage, bufs, acc,
             semg0, semg1, semd0, semd1, sems0, sems1):
    c = lax.axis_index("c")
    s = lax.axis_index("s")
    base = s * ROWS_PER_TILE
    pltpu.sync_copy(srcs.at[pl.ds(base, ROWS_PER_TILE)], src_v)
    pltpu.sync_copy(dsts.at[pl.ds(base, DBLK)], dstage.at[0])
    pltpu.sync_copy(zeros, acc.at[pl.ds(s * ZROWS, ZROWS)])

    @pl.when(s == NS - 1)
    def _():
        pltpu.sync_copy(zeros.at[pl.ds(0, ZTAIL)],
                        acc.at[pl.ds(NS * ZROWS, ZTAIL)])

    plsc.subcore_barrier()

    sems = (semg0, semg1, semd0, semd1, sems0, sems1)

    @pl.when(c == 0)
    def _():
        _sc_pipeline(table0, dsts, base, src_v, dstage, bufs, acc, *sems)

    @pl.when(c == 1)
    def _():
        _sc_pipeline(table1, dsts, base, src_v, dstage, bufs, acc, *sems)

    plsc.subcore_barrier()

    @pl.when(c == 0)
    def _():
        pltpu.sync_copy(acc.at[pl.ds(s * ZROWS, ZROWS)],
                        out0.at[pl.ds(s * ZROWS, ZROWS)])

        @pl.when(s == NS - 1)
        def _():
            pltpu.sync_copy(acc.at[pl.ds(NS * ZROWS, ZTAIL)],
                            out0.at[pl.ds(NS * ZROWS, ZTAIL)])

    @pl.when(c == 1)
    def _():
        pltpu.sync_copy(acc.at[pl.ds(s * ZROWS, ZROWS)],
                        out1.at[pl.ds(s * ZROWS, ZROWS)])

        @pl.when(s == NS - 1)
        def _():
            pltpu.sync_copy(acc.at[pl.ds(NS * ZROWS, ZTAIL)],
                            out1.at[pl.ds(NS * ZROWS, ZTAIL)])


_sc_segsum = functools.partial(
    pl.kernel,
    out_type=(
        jax.ShapeDtypeStruct((N_NODES, HALF), jnp.float32),
        jax.ShapeDtypeStruct((N_NODES, HALF), jnp.float32),
    ),
    mesh=plsc.VectorSubcoreMesh(core_axis_name="c", subcore_axis_name="s"),
    scratch_types=[
        pltpu.VMEM((ROWS_PER_TILE, BATCH), jnp.int32),
        pltpu.VMEM((2, DBLK, BATCH), jnp.int32),
        pltpu.VMEM((2, BATCH, HALF), jnp.float32),
        pltpu.VMEM_SHARED((N_NODES, HALF), jnp.float32),
        pltpu.SemaphoreType.DMA,
        pltpu.SemaphoreType.DMA,
        pltpu.SemaphoreType.DMA,
        pltpu.SemaphoreType.DMA,
        pltpu.SemaphoreType.DMA,
        pltpu.SemaphoreType.DMA,
    ],
)(_sc_body)


def _read_body(s0_ref, s1_ref, bc_ref, wr_ref, br_ref, y_ref, hs_ref):
    hs = jnp.concatenate([s0_ref[...], s1_ref[...]], axis=1) + bc_ref[...]
    hs_ref[...] = hs
    y_ref[...] = jnp.tanh(hs) @ wr_ref[...] + br_ref[...]


def _readout(s0, s1, b_conv2, W_read, b_read2):
    R = 2000
    nb = N_NODES // R
    return pl.pallas_call(
        _read_body,
        grid=(nb,),
        in_specs=[
            pl.BlockSpec((R, HALF), lambda r: (r, 0)),
            pl.BlockSpec((R, HALF), lambda r: (r, 0)),
            pl.BlockSpec((1, D), lambda r: (0, 0)),
            pl.BlockSpec((D, D), lambda r: (0, 0)),
            pl.BlockSpec((1, D), lambda r: (0, 0)),
        ],
        out_specs=[
            pl.BlockSpec((R, D), lambda r: (r, 0)),
            pl.BlockSpec((R, D), lambda r: (r, 0)),
        ],
        out_shape=[
            jax.ShapeDtypeStruct((N_NODES, D), jnp.float32),
            jax.ShapeDtypeStruct((N_NODES, D), jnp.float32),
        ],
    )(s0, s1, b_conv2, W_read, b_read2)


def kernel(x, edge_index, W_emb, b_emb, W_conv, b_conv, W_read, b_read):
    ei = edge_index.astype(jnp.int32)
    srcs2 = ei[0].reshape(ROWS_TOTAL, BATCH)
    dst2 = ei[1].reshape(ROWS_TOTAL, BATCH)
    zeros = jnp.zeros((ZROWS, HALF), jnp.float32)

    t0, t1 = _make_table(x, W_emb, b_emb.reshape(1, D), W_conv)
    s0, s1 = _sc_segsum(t0, t1, srcs2, dst2, zeros)
    y, h_state = _readout(
        s0, s1, b_conv.reshape(1, D), W_read, b_read.reshape(1, D)
    )
    return (y, h_state)


# speedup vs baseline: 8.3864x; 1.0012x over previous
import functools

import jax
import jax.numpy as jnp
from jax import lax
from jax.experimental import pallas as pl
from jax.experimental.pallas import tpu as pltpu
from jax.experimental.pallas import tpu_sc as plsc

N_NODES = 10000
N_EDGES = 160000
D = 256
HALF = 128

NC = 2
NS = 16
BATCH = 125
ROWS_PER_TILE = N_EDGES // NS // BATCH
ROWS_TOTAL = N_EDGES // BATCH
EDGES_PER_TILE = N_EDGES // NS
DBLK = 16
NBLK = ROWS_PER_TILE // DBLK
ZROWS = 624
ZTAIL = N_NODES - NS * ZROWS


def _emb_body(x_ref, we_ref, be_ref, wc_ref, o0_ref, o1_ref):
    wc = we_ref[...] @ wc_ref[...]
    bc = be_ref[...] @ wc_ref[...]
    t = x_ref[...] @ wc + bc
    o0_ref[...] = t[:, :HALF]
    o1_ref[...] = t[:, HALF:]


def _make_table(x, W_emb, b_emb2, W_conv):
    R = 2000
    nb = N_NODES // R
    return pl.pallas_call(
        _emb_body,
        grid=(nb,),
        in_specs=[
            pl.BlockSpec((R, D), lambda r: (r, 0)),
            pl.BlockSpec((D, D), lambda r: (0, 0)),
            pl.BlockSpec((1, D), lambda r: (0, 0)),
            pl.BlockSpec((D, D), lambda r: (0, 0)),
        ],
        out_specs=[
            pl.BlockSpec((R, HALF), lambda r: (r, 0)),
            pl.BlockSpec((R, HALF), lambda r: (r, 0)),
        ],
        out_shape=[
            jax.ShapeDtypeStruct((N_NODES, HALF), jnp.float32),
            jax.ShapeDtypeStruct((N_NODES, HALF), jnp.float32),
        ],
    )(x, W_emb, b_emb2, W_conv)


def _sc_pipeline(table, dsts, base, src_v, dstage, bufs, acc,
                 semg0, semg1, semd0, semd1, sems0, sems1):
    pltpu.async_copy(table.at[src_v.at[0]], bufs.at[0], semg0)

    def step(i, carry):
        p = i % 2
        q = (i + 1) % 2
        blk = i // DBLK
        bp = blk % 2

        @pl.when(i >= 1)
        def _():
            dummy = acc.at[dstage.at[bp, 0]]

            @pl.when(q == 0)
            def _():
                pltpu.make_async_copy(bufs.at[0], dummy, sems0).wait()

            @pl.when(q == 1)
            def _():
                pltpu.make_async_copy(bufs.at[1], dummy, sems1).wait()

        @pl.when((i % DBLK == 0) & (blk < NBLK - 1))
        def _():
            nxt = dsts.at[pl.ds(base + (blk + 1) * DBLK, DBLK)]

            @pl.when(bp == 0)
            def _():
                pltpu.async_copy(nxt, dstage.at[1], semd1)

            @pl.when(bp == 1)
            def _():
                pltpu.async_copy(nxt, dstage.at[0], semd0)

        @pl.when((i % DBLK == 0) & (i > 0))
        def _():
            cur = dsts.at[pl.ds(base + blk * DBLK, DBLK)]

            @pl.when(bp == 0)
            def _():
                pltpu.make_async_copy(cur, dstage.at[0], semd0).wait()

            @pl.when(bp == 1)
            def _():
                pltpu.make_async_copy(cur, dstage.at[1], semd1).wait()

        @pl.when(i < ROWS_PER_TILE - 1)
        def _():
            gsrc = table.at[src_v.at[i + 1]]

            @pl.when(q == 0)
            def _():
                pltpu.async_copy(gsrc, bufs.at[0], semg0)

            @pl.when(q == 1)
            def _():
                pltpu.async_copy(gsrc, bufs.at[1], semg1)

        gcur = table.at[src_v.at[i]]

        @pl.when(p == 0)
        def _():
            pltpu.make_async_copy(gcur, bufs.at[0], semg0).wait()

        @pl.when(p == 1)
        def _():
            pltpu.make_async_copy(gcur, bufs.at[1], semg1).wait()

        sdst = acc.at[dstage.at[bp, i % DBLK]]

        @pl.when(p == 0)
        def _():
            pltpu.async_copy(bufs.at[0], sdst, sems0, add=True)

        @pl.when(p == 1)
        def _():
            pltpu.async_copy(bufs.at[1], sdst, sems1, add=True)

        return carry

    lax.fori_loop(0, ROWS_PER_TILE, step, 0)
    lastd = acc.at[dstage.at[(NBLK - 1) % 2, 0]]
    pltpu.make_async_copy(bufs.at[1], lastd, sems1).wait()


def _sc_body(table0, table1, srcs, dsts, zeros, out0, out1,
             src_v, dstage, bufs, acc,
             semg0, semg1, semd0, semd1, sems0, sems1):
    c = lax.axis_index("c")
    s = lax.axis_index("s")
    base = s * ROWS_PER_TILE
    pltpu.sync_copy(srcs.at[pl.ds(base, ROWS_PER_TILE)], src_v)
    pltpu.sync_copy(dsts.at[pl.ds(base, DBLK)], dstage.at[0])
    pltpu.sync_copy(zeros, acc.at[pl.ds(s * ZROWS, ZROWS)])

    @pl.when(s == NS - 1)
    def _():
        pltpu.sync_copy(zeros.at[pl.ds(0, ZTAIL)],
                        acc.at[pl.ds(NS * ZROWS, ZTAIL)])

    plsc.subcore_barrier()

    sems = (semg0, semg1, semd0, semd1, sems0, sems1)

    @pl.when(c == 0)
    def _():
        _sc_pipeline(table0, dsts, base, src_v, dstage, bufs, acc, *sems)

    @pl.when(c == 1)
    def _():
        _sc_pipeline(table1, dsts, base, src_v, dstage, bufs, acc, *sems)

    plsc.subcore_barrier()

    @pl.when(c == 0)
    def _():
        pltpu.sync_copy(acc.at[pl.ds(s * ZROWS, ZROWS)],
                        out0.at[pl.ds(s * ZROWS, ZROWS)])

        @pl.when(s == NS - 1)
        def _():
            pltpu.sync_copy(acc.at[pl.ds(NS * ZROWS, ZTAIL)],
                            out0.at[pl.ds(NS * ZROWS, ZTAIL)])

    @pl.when(c == 1)
    def _():
        pltpu.sync_copy(acc.at[pl.ds(s * ZROWS, ZROWS)],
                        out1.at[pl.ds(s * ZROWS, ZROWS)])

        @pl.when(s == NS - 1)
        def _():
            pltpu.sync_copy(acc.at[pl.ds(NS * ZROWS, ZTAIL)],
                            out1.at[pl.ds(NS * ZROWS, ZTAIL)])


_sc_segsum = functools.partial(
    pl.kernel,
    out_type=(
        jax.ShapeDtypeStruct((N_NODES, HALF), jnp.float32),
        jax.ShapeDtypeStruct((N_NODES, HALF), jnp.float32),
    ),
    mesh=plsc.VectorSubcoreMesh(core_axis_name="c", subcore_axis_name="s"),
    scratch_types=[
        pltpu.VMEM((ROWS_PER_TILE, BATCH), jnp.int32),
        pltpu.VMEM((2, DBLK, BATCH), jnp.int32),
        pltpu.VMEM((2, BATCH, HALF), jnp.float32),
        pltpu.VMEM_SHARED((N_NODES, HALF), jnp.float32),
        pltpu.SemaphoreType.DMA,
        pltpu.SemaphoreType.DMA,
        pltpu.SemaphoreType.DMA,
        pltpu.SemaphoreType.DMA,
        pltpu.SemaphoreType.DMA,
        pltpu.SemaphoreType.DMA,
    ],
)(_sc_body)


def _read_body(s0_ref, s1_ref, bc_ref, wr_ref, br_ref, y_ref, hs_ref):
    hs = jnp.concatenate([s0_ref[...], s1_ref[...]], axis=1) + bc_ref[...]
    hs_ref[...] = hs
    y_ref[...] = jnp.tanh(hs) @ wr_ref[...] + br_ref[...]


def _readout(s0, s1, b_conv2, W_read, b_read2):
    R = 2000
    nb = N_NODES // R
    return pl.pallas_call(
        _read_body,
        grid=(nb,),
        in_specs=[
            pl.BlockSpec((R, HALF), lambda r: (r, 0)),
            pl.BlockSpec((R, HALF), lambda r: (r, 0)),
            pl.BlockSpec((1, D), lambda r: (0, 0)),
            pl.BlockSpec((D, D), lambda r: (0, 0)),
            pl.BlockSpec((1, D), lambda r: (0, 0)),
        ],
        out_specs=[
            pl.BlockSpec((R, D), lambda r: (r, 0)),
            pl.BlockSpec((R, D), lambda r: (r, 0)),
        ],
        out_shape=[
            jax.ShapeDtypeStruct((N_NODES, D), jnp.float32),
            jax.ShapeDtypeStruct((N_NODES, D), jnp.float32),
        ],
    )(s0, s1, b_conv2, W_read, b_read2)


def kernel(x, edge_index, W_emb, b_emb, W_conv, b_conv, W_read, b_read):
    ei = edge_index.astype(jnp.int32)
    srcs2 = ei[0].reshape(ROWS_TOTAL, BATCH)
    dst2 = ei[1].reshape(ROWS_TOTAL, BATCH)
    zeros = jnp.zeros((ZROWS, HALF), jnp.float32)

    t0, t1 = _make_table(x, W_emb, b_emb.reshape(1, D), W_conv)
    s0, s1 = _sc_segsum(t0, t1, srcs2, dst2, zeros)
    y, h_state = _readout(
        s0, s1, b_conv.reshape(1, D), W_read, b_read.reshape(1, D)
    )
    return (y, h_state)


# speedup vs baseline: 8.3893x; 1.0004x over previous
import functools

import jax
import jax.numpy as jnp
from jax import lax
from jax.experimental import pallas as pl
from jax.experimental.pallas import tpu as pltpu
from jax.experimental.pallas import tpu_sc as plsc

N_NODES = 10000
N_EDGES = 160000
D = 256
HALF = 128

NS = 16
BATCH = 125
ROWS_PER_TILE = N_EDGES // NS // BATCH
ROWS_TOTAL = N_EDGES // BATCH
DBLK = 16
NBLK = ROWS_PER_TILE // DBLK
ZROWS = 624
ZTAIL = N_NODES - NS * ZROWS


def _emb_body(x_ref, we_ref, be_ref, wc_ref, o0_ref, o1_ref):
    wc = we_ref[...] @ wc_ref[...]
    bc = be_ref[...] @ wc_ref[...]
    t = x_ref[...] @ wc + bc
    o0_ref[...] = t[:, :HALF]
    o1_ref[...] = t[:, HALF:]


def _make_table(x, W_emb, b_emb2, W_conv):
    R = 2000
    nb = N_NODES // R
    return pl.pallas_call(
        _emb_body,
        grid=(nb,),
        in_specs=[
            pl.BlockSpec((R, D), lambda r: (r, 0)),
            pl.BlockSpec((D, D), lambda r: (0, 0)),
            pl.BlockSpec((1, D), lambda r: (0, 0)),
            pl.BlockSpec((D, D), lambda r: (0, 0)),
        ],
        out_specs=[
            pl.BlockSpec((R, HALF), lambda r: (r, 0)),
            pl.BlockSpec((R, HALF), lambda r: (r, 0)),
        ],
        out_shape=[
            jax.ShapeDtypeStruct((N_NODES, HALF), jnp.float32),
            jax.ShapeDtypeStruct((N_NODES, HALF), jnp.float32),
        ],
    )(x, W_emb, b_emb2, W_conv)


def _sc_pipeline(table, dsts, base, src_v, dstage, bufs, acc,
                 semg0, semg1, semd0, semd1, sems0, sems1):
    pltpu.async_copy(table.at[src_v.at[0]], bufs.at[0], semg0)

    def step(i, carry):
        p = i % 2
        q = (i + 1) % 2
        blk = i // DBLK
        bp = blk % 2

        @pl.when(i >= 1)
        def _():
            dummy = acc.at[dstage.at[bp, 0]]

            @pl.when(q == 0)
            def _():
                pltpu.make_async_copy(bufs.at[0], dummy, sems0).wait()

            @pl.when(q == 1)
            def _():
                pltpu.make_async_copy(bufs.at[1], dummy, sems1).wait()

        @pl.when((i % DBLK == 0) & (blk < NBLK - 1))
        def _():
            nxt = dsts.at[pl.ds(base + (blk + 1) * DBLK, DBLK)]

            @pl.when(bp == 0)
            def _():
                pltpu.async_copy(nxt, dstage.at[1], semd1)

            @pl.when(bp == 1)
            def _():
                pltpu.async_copy(nxt, dstage.at[0], semd0)

        @pl.when((i % DBLK == 0) & (i > 0))
        def _():
            cur = dsts.at[pl.ds(base + blk * DBLK, DBLK)]

            @pl.when(bp == 0)
            def _():
                pltpu.make_async_copy(cur, dstage.at[0], semd0).wait()

            @pl.when(bp == 1)
            def _():
                pltpu.make_async_copy(cur, dstage.at[1], semd1).wait()

        @pl.when(i < ROWS_PER_TILE - 1)
        def _():
            gsrc = table.at[src_v.at[i + 1]]

            @pl.when(q == 0)
            def _():
                pltpu.async_copy(gsrc, bufs.at[0], semg0)

            @pl.when(q == 1)
            def _():
                pltpu.async_copy(gsrc, bufs.at[1], semg1)

        gcur = table.at[src_v.at[i]]

        @pl.when(p == 0)
        def _():
            pltpu.make_async_copy(gcur, bufs.at[0], semg0).wait()

        @pl.when(p == 1)
        def _():
            pltpu.make_async_copy(gcur, bufs.at[1], semg1).wait()

        sdst = acc.at[dstage.at[bp, i % DBLK]]

        @pl.when(p == 0)
        def _():
            pltpu.async_copy(bufs.at[0], sdst, sems0, add=True)

        @pl.when(p == 1)
        def _():
            pltpu.async_copy(bufs.at[1], sdst, sems1, add=True)

        return carry

    lax.fori_loop(0, ROWS_PER_TILE, step, 0)
    lastd = acc.at[dstage.at[(NBLK - 1) % 2, 0]]
    pltpu.make_async_copy(bufs.at[1], lastd, sems1).wait()


def _sc_body(table0, table1, srcs, dsts, zeros, out0, out1,
             src_v, dstage, bufs, acc,
             semg0, semg1, semd0, semd1, sems0, sems1):
    c = lax.axis_index("c")
    s = lax.axis_index("s")
    base = s * ROWS_PER_TILE
    pltpu.sync_copy(srcs.at[pl.ds(base, ROWS_PER_TILE)], src_v)
    pltpu.sync_copy(dsts.at[pl.ds(base, DBLK)], dstage.at[0])
    pltpu.sync_copy(zeros, acc.at[pl.ds(s * ZROWS, ZROWS)])

    @pl.when(s == NS - 1)
    def _():
        pltpu.sync_copy(zeros.at[pl.ds(0, ZTAIL)],
                        acc.at[pl.ds(NS * ZROWS, ZTAIL)])

    plsc.subcore_barrier()

    sems = (semg0, semg1, semd0, semd1, sems0, sems1)

    @pl.when(c == 0)
    def _():
        _sc_pipeline(table0, dsts, base, src_v, dstage, bufs, acc, *sems)

    @pl.when(c == 1)
    def _():
        _sc_pipeline(table1, dsts, base, src_v, dstage, bufs, acc, *sems)

    plsc.subcore_barrier()

    @pl.when(c == 0)
    def _():
        pltpu.sync_copy(acc.at[pl.ds(s * ZROWS, ZROWS)],
                        out0.at[pl.ds(s * ZROWS, ZROWS)])

        @pl.when(s == NS - 1)
        def _():
            pltpu.sync_copy(acc.at[pl.ds(NS * ZROWS, ZTAIL)],
                            out0.at[pl.ds(NS * ZROWS, ZTAIL)])

    @pl.when(c == 1)
    def _():
        pltpu.sync_copy(acc.at[pl.ds(s * ZROWS, ZROWS)],
                        out1.at[pl.ds(s * ZROWS, ZROWS)])

        @pl.when(s == NS - 1)
        def _():
            pltpu.sync_copy(acc.at[pl.ds(NS * ZROWS, ZTAIL)],
                            out1.at[pl.ds(NS * ZROWS, ZTAIL)])


_sc_segsum = functools.partial(
    pl.kernel,
    out_type=(
        jax.ShapeDtypeStruct((N_NODES, HALF), jnp.float32),
        jax.ShapeDtypeStruct((N_NODES, HALF), jnp.float32),
    ),
    mesh=plsc.VectorSubcoreMesh(core_axis_name="c", subcore_axis_name="s"),
    scratch_types=[
        pltpu.VMEM((ROWS_PER_TILE, BATCH), jnp.int32),
        pltpu.VMEM((2, DBLK, BATCH), jnp.int32),
        pltpu.VMEM((2, BATCH, HALF), jnp.float32),
        pltpu.VMEM_SHARED((N_NODES, HALF), jnp.float32),
        pltpu.SemaphoreType.DMA,
        pltpu.SemaphoreType.DMA,
        pltpu.SemaphoreType.DMA,
        pltpu.SemaphoreType.DMA,
        pltpu.SemaphoreType.DMA,
        pltpu.SemaphoreType.DMA,
    ],
)(_sc_body)


def _read_body(s0_ref, s1_ref, bc_ref, wr_ref, br_ref, y_ref, hs_ref):
    hs = jnp.concatenate([s0_ref[...], s1_ref[...]], axis=1) + bc_ref[...]
    hs_ref[...] = hs
    y_ref[...] = jnp.tanh(hs) @ wr_ref[...] + br_ref[...]


def _readout(s0, s1, b_conv2, W_read, b_read2):
    R = 2000
    nb = N_NODES // R
    return pl.pallas_call(
        _read_body,
        grid=(nb,),
        in_specs=[
            pl.BlockSpec((R, HALF), lambda r: (r, 0)),
            pl.BlockSpec((R, HALF), lambda r: (r, 0)),
            pl.BlockSpec((1, D), lambda r: (0, 0)),
            pl.BlockSpec((D, D), lambda r: (0, 0)),
            pl.BlockSpec((1, D), lambda r: (0, 0)),
        ],
        out_specs=[
            pl.BlockSpec((R, D), lambda r: (r, 0)),
            pl.BlockSpec((R, D), lambda r: (r, 0)),
        ],
        out_shape=[
            jax.ShapeDtypeStruct((N_NODES, D), jnp.float32),
            jax.ShapeDtypeStruct((N_NODES, D), jnp.float32),
        ],
    )(s0, s1, b_conv2, W_read, b_read2)


def kernel(x, edge_index, W_emb, b_emb, W_conv, b_conv, W_read, b_read):
    ei = edge_index.astype(jnp.int32)
    srcs2 = ei[0].reshape(ROWS_TOTAL, BATCH)
    dst2 = ei[1].reshape(ROWS_TOTAL, BATCH)
    zeros = jnp.zeros((ZROWS, HALF), jnp.float32)

    t0, t1 = _make_table(x, W_emb, b_emb.reshape(1, D), W_conv)
    s0, s1 = _sc_segsum(t0, t1, srcs2, dst2, zeros)
    y, h_state = _readout(
        s0, s1, b_conv.reshape(1, D), W_read, b_read.reshape(1, D)
    )
    return (y, h_state)
